# baseline (speedup 1.0000x reference)
import functools
import math

import jax
import jax.numpy as jnp
from jax import lax
from jax.experimental import pallas as pl
from jax.experimental.pallas import tpu as pltpu
from jax.experimental.pallas import tpu_sc as plsc

D_EMB = 128
SEQ = 200
BATCH = 1024
NUM_CORES = 2
NUM_SUBCORES = 16
NW = NUM_CORES * NUM_SUBCORES
TOK_PER_W = BATCH * SEQ // NW
WIN = 128
NWIN = TOK_PER_W // WIN
LANES = 16
SCALE = math.sqrt(float(D_EMB))
NBUF = 5
LOOKAHEAD = 2


def kernel(x, table, pe):
    B, S = x.shape
    V, D = table.shape
    assert (B, S, D) == (BATCH, SEQ, D_EMB)
    xf = x.reshape(B * S).astype(jnp.int32)

    mesh = plsc.VectorSubcoreMesh(core_axis_name="c", subcore_axis_name="s")

    @functools.partial(
        pl.kernel,
        out_type=jax.ShapeDtypeStruct((B * S, D), jnp.float32),
        mesh=mesh,
        scratch_types=[
            pltpu.VMEM((TOK_PER_W,), jnp.int32),
            pltpu.VMEM((SEQ, D_EMB), jnp.float32),
            pltpu.VMEM((NBUF, WIN, D_EMB), jnp.float32),
            pltpu.SemaphoreType.DMA,
            pltpu.SemaphoreType.DMA,
            pltpu.SemaphoreType.DMA,
            pltpu.SemaphoreType.DMA,
            pltpu.SemaphoreType.DMA,
            pltpu.SemaphoreType.DMA,
            pltpu.SemaphoreType.DMA,
            pltpu.SemaphoreType.DMA,
            pltpu.SemaphoreType.DMA,
            pltpu.SemaphoreType.DMA,
            pltpu.SemaphoreType.DMA,
            pltpu.SemaphoreType.DMA,
        ],
    )
    def emb_kernel(table_hbm, xf_hbm, pe_hbm, out_hbm, idx_v, pe_v, ring,
                   psem, pesem, g0, g1, g2, g3, g4, w0, w1, w2, w3, w4):
        wid = lax.axis_index("s") * NUM_CORES + lax.axis_index("c")
        tbase = wid * TOK_PER_W
        gsem = (g0, g1, g2, g3, g4)
        wsem = (w0, w1, w2, w3, w4)

        idx_cp = pltpu.make_async_copy(
            xf_hbm.at[pl.ds(tbase, TOK_PER_W)], idx_v, psem)
        pe_cp = pltpu.make_async_copy(pe_hbm.at[pl.ds(0, SEQ)], pe_v, pesem)
        idx_cp.start()
        pe_cp.start()
        idx_cp.wait()
        pe_started = [True]

        def gather_copy(w, b):
            return pltpu.make_async_copy(
                table_hbm.at[idx_v.at[pl.ds(w * WIN, WIN)]],
                ring.at[b], gsem[b])

        def write_copy(w, b):
            return pltpu.make_async_copy(
                ring.at[b], out_hbm.at[pl.ds(tbase + w * WIN, WIN)], wsem[b])

        def write_wait(b):
            pltpu.make_async_copy(
                ring.at[b], out_hbm.at[pl.ds(0, WIN)], wsem[b]).wait()

        def compute(w, b):
            buf = ring.at[b]
            p0 = lax.rem(w * WIN, SEQ)
            len1 = jnp.minimum(SEQ - p0, WIN)

            @pl.loop(0, len1)
            def _tok_a(j):
                for c in range(D_EMB // LANES):
                    sl = pl.ds(c * LANES, LANES)
                    buf[j, sl] = buf[j, sl] * SCALE + pe_v[p0 + j, sl]

            @pl.loop(len1, WIN)
            def _tok_b(j):
                for c in range(D_EMB // LANES):
                    sl = pl.ds(c * LANES, LANES)
                    buf[j, sl] = buf[j, sl] * SCALE + pe_v[j - len1, sl]

        def substep(w, b, prefetch_wait, guard_tail):
            nb = (b + LOOKAHEAD) % NBUF

            def _prefetch():
                if prefetch_wait:
                    write_wait(nb)
                gather_copy(w + LOOKAHEAD, nb).start()

            if guard_tail:
                pl.when(w + LOOKAHEAD < NWIN)(_prefetch)
            else:
                _prefetch()

            gather_copy(w, b).wait()
            if pe_started:
                pe_cp.wait()
                pe_started.clear()
            compute(w, b)
            write_copy(w, b).start()

        gather_copy(0, 0).start()
        gather_copy(1, 1).start()

        substep(0, 0, prefetch_wait=False, guard_tail=False)
        substep(1, 1, prefetch_wait=False, guard_tail=False)
        substep(2, 2, prefetch_wait=False, guard_tail=False)
        substep(3, 3, prefetch_wait=True, guard_tail=False)
        substep(4, 4, prefetch_wait=True, guard_tail=False)

        @pl.loop(1, NWIN // NBUF)
        def _grp(g):
            base = NBUF * g
            substep(base, 0, prefetch_wait=True, guard_tail=True)
            substep(base + 1, 1, prefetch_wait=True, guard_tail=True)
            substep(base + 2, 2, prefetch_wait=True, guard_tail=True)
            substep(base + 3, 3, prefetch_wait=True, guard_tail=True)
            substep(base + 4, 4, prefetch_wait=True, guard_tail=True)

        for b in range(NBUF):
            write_wait(b)

    out = emb_kernel(table, xf, pe)
    return out.reshape(B, S, D)


# speedup vs baseline: 1.4992x; 1.4992x over previous
import functools
import math

import jax
import jax.numpy as jnp
from jax import lax
from jax.experimental import pallas as pl
from jax.experimental.pallas import tpu as pltpu
from jax.experimental.pallas import tpu_sc as plsc

D_EMB = 128
SEQ = 200
BATCH = 1024
NUM_CORES = 2
NUM_SUBCORES = 16
NW = NUM_CORES * NUM_SUBCORES
ROWS_PER_W = BATCH // NW
LANES = 16
SCALE = math.sqrt(float(D_EMB))
GATHER_SPLITS = ((0, 128), (128, 72))


def kernel(x, table, pe):
    B, S = x.shape
    V, D = table.shape
    assert (B, S, D) == (BATCH, SEQ, D_EMB)
    xf = x.reshape(B * S).astype(jnp.int32)
    pe_s = pe[:S]

    mesh = plsc.VectorSubcoreMesh(core_axis_name="c", subcore_axis_name="s")

    @functools.partial(
        pl.kernel,
        out_type=jax.ShapeDtypeStruct((B * S, D), jnp.float32),
        mesh=mesh,
        scratch_types=[
            pltpu.VMEM((ROWS_PER_W * SEQ,), jnp.int32),
            pltpu.VMEM((SEQ, D_EMB), jnp.float32),
            pltpu.VMEM((SEQ, D_EMB), jnp.float32),
            pltpu.SemaphoreType.DMA,
        ],
    )
    def emb_kernel(table_hbm, xf_hbm, pe_hbm, out_hbm, idx_v, pe_v, rows_v, gsem):
        wid = lax.axis_index("s") * NUM_CORES + lax.axis_index("c")

        pltpu.sync_copy(xf_hbm.at[pl.ds(wid * (ROWS_PER_W * SEQ), ROWS_PER_W * SEQ)],
                        idx_v)
        pltpu.sync_copy(pe_hbm, pe_v)

        def gather_copies(r):
            for off, win in GATHER_SPLITS:
                yield pltpu.make_async_copy(
                    table_hbm.at[idx_v.at[pl.ds(r * SEQ + off, win)]],
                    rows_v.at[pl.ds(off, win)],
                    gsem,
                )

        @pl.loop(0, ROWS_PER_W)
        def _row(r):
            for c in gather_copies(r):
                c.start()
            for c in gather_copies(r):
                c.wait()

            @pl.loop(0, SEQ)
            def _tok(i):
                for c in range(D_EMB // LANES):
                    sl = pl.ds(c * LANES, LANES)
                    rows_v[i, sl] = rows_v[i, sl] * SCALE + pe_v[i, sl]

            pltpu.sync_copy(
                rows_v, out_hbm.at[pl.ds((wid * ROWS_PER_W + r) * SEQ, SEQ)])

    out = emb_kernel(table, xf, pe_s)
    return out.reshape(B, S, D)


# speedup vs baseline: 2.2349x; 1.4907x over previous
import functools
import math

import jax
import jax.numpy as jnp
from jax import lax
from jax.experimental import pallas as pl
from jax.experimental.pallas import tpu as pltpu
from jax.experimental.pallas import tpu_sc as plsc

D_EMB = 128
SEQ = 200
BATCH = 1024
NUM_CORES = 2
NUM_SUBCORES = 16
NW = NUM_CORES * NUM_SUBCORES
ROWS_PER_W = BATCH // NW
LANES = 16
SCALE = math.sqrt(float(D_EMB))
GATHER_SPLITS = ((0, 128), (128, 72))


def kernel(x, table, pe):
    B, S = x.shape
    V, D = table.shape
    assert (B, S, D) == (BATCH, SEQ, D_EMB)
    xf = x.reshape(B * S).astype(jnp.int32)
    pe_s = pe[:S]

    mesh = plsc.VectorSubcoreMesh(core_axis_name="c", subcore_axis_name="s")

    @functools.partial(
        pl.kernel,
        out_type=jax.ShapeDtypeStruct((B * S, D), jnp.float32),
        mesh=mesh,
        scratch_types=[
            pltpu.VMEM((ROWS_PER_W * SEQ,), jnp.int32),
            pltpu.VMEM((SEQ, D_EMB), jnp.float32),
            pltpu.VMEM((SEQ, D_EMB), jnp.float32),
            pltpu.VMEM((SEQ, D_EMB), jnp.float32),
            pltpu.SemaphoreType.DMA,
            pltpu.SemaphoreType.DMA,
        ],
    )
    def emb_kernel(table_hbm, xf_hbm, pe_hbm, out_hbm, idx_v, pe_v,
                   rows0, rows1, g0, g1):
        wid = lax.axis_index("s") * NUM_CORES + lax.axis_index("c")
        rows = (rows0, rows1)
        gsem = (g0, g1)

        pltpu.sync_copy(xf_hbm.at[pl.ds(wid * (ROWS_PER_W * SEQ), ROWS_PER_W * SEQ)],
                        idx_v)
        pltpu.sync_copy(pe_hbm, pe_v)

        def gather_copies(r, b):
            for off, win in GATHER_SPLITS:
                yield pltpu.make_async_copy(
                    table_hbm.at[idx_v.at[pl.ds(r * SEQ + off, win)]],
                    rows[b].at[pl.ds(off, win)],
                    gsem[b],
                )

        def gather_start(r, b):
            for c in gather_copies(r, b):
                c.start()

        def gather_wait(r, b):
            for c in gather_copies(r, b):
                c.wait()

        def substep(r, b):
            @pl.when(r + 1 < ROWS_PER_W)
            def _():
                gather_start(r + 1, 1 - b)

            gather_wait(r, b)

            buf = rows[b]

            @pl.loop(0, SEQ)
            def _tok(i):
                for c in range(D_EMB // LANES):
                    sl = pl.ds(c * LANES, LANES)
                    buf[i, sl] = buf[i, sl] * SCALE + pe_v[i, sl]

            pltpu.sync_copy(
                buf, out_hbm.at[pl.ds((wid * ROWS_PER_W + r) * SEQ, SEQ)])

        gather_start(0, 0)

        @pl.loop(0, ROWS_PER_W // 2)
        def _grp(g):
            substep(2 * g, 0)
            substep(2 * g + 1, 1)

    out = emb_kernel(table, xf, pe_s)
    return out.reshape(B, S, D)


# speedup vs baseline: 2.2505x; 1.0070x over previous
import functools
import math

import jax
import jax.numpy as jnp
from jax import lax
from jax.experimental import pallas as pl
from jax.experimental.pallas import tpu as pltpu
from jax.experimental.pallas import tpu_sc as plsc

D_EMB = 128
SEQ = 200
BATCH = 1024
NUM_CORES = 2
NUM_SUBCORES = 16
NW = NUM_CORES * NUM_SUBCORES
ROWS_PER_W = BATCH // NW
LANES = 16
SCALE = math.sqrt(float(D_EMB))
GATHER_SPLITS = ((0, 128), (128, 72))


def kernel(x, table, pe):
    B, S = x.shape
    V, D = table.shape
    assert (B, S, D) == (BATCH, SEQ, D_EMB)
    xf = x.reshape(B * S).astype(jnp.int32)
    pe_s = pe[:S]

    mesh = plsc.VectorSubcoreMesh(core_axis_name="c", subcore_axis_name="s")

    @functools.partial(
        pl.kernel,
        out_type=jax.ShapeDtypeStruct((B * S, D), jnp.float32),
        mesh=mesh,
        scratch_types=[
            pltpu.VMEM((ROWS_PER_W * SEQ,), jnp.int32),
            pltpu.VMEM((SEQ, D_EMB), jnp.float32),
            pltpu.VMEM((SEQ, D_EMB), jnp.float32),
            pltpu.VMEM((SEQ, D_EMB), jnp.float32),
            pltpu.SemaphoreType.DMA,
            pltpu.SemaphoreType.DMA,
            pltpu.SemaphoreType.DMA,
            pltpu.SemaphoreType.DMA,
        ],
    )
    def emb_kernel(table_hbm, xf_hbm, pe_hbm, out_hbm, idx_v, pe_v,
                   rows0, rows1, g0, g1, w0, w1):
        wid = lax.axis_index("s") * NUM_CORES + lax.axis_index("c")
        rows = (rows0, rows1)
        gsem = (g0, g1)
        wsem = (w0, w1)

        pltpu.sync_copy(xf_hbm.at[pl.ds(wid * (ROWS_PER_W * SEQ), ROWS_PER_W * SEQ)],
                        idx_v)
        pltpu.sync_copy(pe_hbm, pe_v)

        def gather_copies(r, b):
            for off, win in GATHER_SPLITS:
                yield pltpu.make_async_copy(
                    table_hbm.at[idx_v.at[pl.ds(r * SEQ + off, win)]],
                    rows[b].at[pl.ds(off, win)],
                    gsem[b],
                )

        def gather_start(r, b):
            for c in gather_copies(r, b):
                c.start()

        def gather_wait(r, b):
            for c in gather_copies(r, b):
                c.wait()

        def write_start(r, b):
            pltpu.async_copy(
                rows[b], out_hbm.at[pl.ds((wid * ROWS_PER_W + r) * SEQ, SEQ)],
                wsem[b])

        def write_wait(b):
            pltpu.make_async_copy(
                rows[b], out_hbm.at[pl.ds(0, SEQ)], wsem[b]).wait()

        def substep(r, b, first=False):
            nb = 1 - b

            @pl.when(r + 1 < ROWS_PER_W)
            def _():
                if not first:
                    write_wait(nb)
                gather_start(r + 1, nb)

            gather_wait(r, b)

            buf = rows[b]

            @pl.loop(0, SEQ)
            def _tok(i):
                for c in range(D_EMB // LANES):
                    sl = pl.ds(c * LANES, LANES)
                    buf[i, sl] = buf[i, sl] * SCALE + pe_v[i, sl]

            write_start(r, b)

        gather_start(0, 0)
        substep(0, 0, first=True)

        @pl.loop(0, (ROWS_PER_W - 2) // 2)
        def _grp(g):
            substep(2 * g + 1, 1)
            substep(2 * g + 2, 0)

        substep(ROWS_PER_W - 1, 1)

        write_wait(0)
        write_wait(1)

    out = emb_kernel(table, xf, pe_s)
    return out.reshape(B, S, D)
